# baseline (speedup 1.0000x reference)
import functools

import jax
import jax.numpy as jnp
import numpy as np
from jax import lax
from jax.experimental import pallas as pl
from jax.experimental.pallas import tpu as pltpu
from jax.experimental.pallas import tpu_sc as plsc

N = 10000
E = 320000
D = 128
G = 64
C = 2
EPS = 1e-5

NC = 2
NS = 16
NW = NC * NS
EPT = E // NW
EPT_P = 10240
CH = 128
NCH = EPT_P // CH
NP = NCH // 2
NPAD = 10240
RPT = NPAD // NS
ZR = 8

BN = 1000
NB = N // BN


def _deg_body(dst_hbm, out_hbm, dst_v, hist_v):
    c = lax.axis_index("c")
    s = lax.axis_index("s")
    wid = c * NS + s
    pltpu.sync_copy(dst_hbm.at[wid], dst_v)
    z16 = jnp.zeros((16,), jnp.float32)

    def zb(i, carry):
        hist_v[0, pl.ds(i * 16, 16)] = z16
        return carry

    lax.fori_loop(0, N // 16, zb, 0)
    ones16 = jnp.ones((16,), jnp.float32)
    zeros_i = jnp.zeros((16,), jnp.int32)

    def body(i, carry):
        idx = dst_v[i]
        plsc.addupdate_scatter(hist_v, [zeros_i, idx], ones16)
        return carry

    lax.fori_loop(0, EPT // 16, body, 0)
    pltpu.sync_copy(hist_v, out_hbm.at[wid])


def _msg_body(h_hbm, comb_hbm, out_hbm, idx4a, idx4b, rows_a, rows_b,
              zbuf, acc_sh, sem_i, sem_i2, sem_a, sem_b, sem_sa, sem_sb):
    c = lax.axis_index("c")
    s = lax.axis_index("s")
    wid = c * NS + s
    z16 = jnp.zeros((16,), jnp.float32)

    def zb(i, carry):
        zbuf[i // 8, pl.ds((i % 8) * 16, 16)] = z16
        return carry

    lax.fori_loop(0, ZR * (D // 16), zb, 0)

    def za(k, carry):
        pltpu.sync_copy(zbuf, acc_sh.at[pl.ds(s * RPT + k * ZR, ZR)])
        return carry

    lax.fori_loop(0, RPT // ZR, za, 0)
    plsc.subcore_barrier()

    pltpu.async_copy(comb_hbm.at[wid, 0], idx4a, sem_i)
    pltpu.async_copy(comb_hbm.at[wid, 1], idx4b, sem_i2)

    def body(j, carry):
        p = 2 * j
        pltpu.make_async_copy(comb_hbm.at[wid, 0], idx4a, sem_i).wait()
        da = pltpu.async_copy(h_hbm.at[idx4a.at[0]], rows_a, sem_a)
        db = pltpu.async_copy(h_hbm.at[idx4a.at[1]], rows_b, sem_b)
        da.wait()
        sa = pltpu.async_copy(rows_a, acc_sh.at[idx4a.at[2]], sem_sa,
                              add=True)
        db.wait()
        sb = pltpu.async_copy(rows_b, acc_sh.at[idx4a.at[3]], sem_sb,
                              add=True)
        pltpu.make_async_copy(comb_hbm.at[wid, 1], idx4b, sem_i2).wait()
        sa.wait()
        da2 = pltpu.async_copy(h_hbm.at[idx4b.at[0]], rows_a, sem_a)
        sb.wait()
        db2 = pltpu.async_copy(h_hbm.at[idx4b.at[1]], rows_b, sem_b)

        @pl.when(p + 2 < NP)
        def _():
            pltpu.async_copy(comb_hbm.at[wid, p + 2], idx4a, sem_i)

        da2.wait()
        sa2 = pltpu.async_copy(rows_a, acc_sh.at[idx4b.at[2]], sem_sa,
                               add=True)
        db2.wait()
        sb2 = pltpu.async_copy(rows_b, acc_sh.at[idx4b.at[3]], sem_sb,
                               add=True)
        sa2.wait()
        sb2.wait()

        @pl.when(p + 3 < NP)
        def _():
            pltpu.async_copy(comb_hbm.at[wid, p + 3], idx4b, sem_i2)

        return carry

    lax.fori_loop(0, NP // 2, body, 0)
    plsc.subcore_barrier()
    pltpu.sync_copy(acc_sh.at[pl.ds(s * RPT, RPT)],
                    out_hbm.at[c, pl.ds(s * RPT, RPT)])


@functools.lru_cache(maxsize=None)
def _sc_kernels():
    mesh = plsc.VectorSubcoreMesh(core_axis_name="c", subcore_axis_name="s",
                                  num_cores=NC, num_subcores=NS)
    deg = pl.kernel(
        _deg_body,
        out_type=jax.ShapeDtypeStruct((NW, 1, N), jnp.float32),
        mesh=mesh,
        scratch_types=[
            pltpu.VMEM((EPT // 16, 16), jnp.int32),
            pltpu.VMEM((1, N), jnp.float32),
        ],
        compiler_params=pltpu.CompilerParams(needs_layout_passes=False),
    )
    msg = pl.kernel(
        _msg_body,
        out_type=jax.ShapeDtypeStruct((NC, NPAD, D), jnp.float32),
        mesh=mesh,
        scratch_types=[
            pltpu.VMEM((4, CH), jnp.int32),
            pltpu.VMEM((4, CH), jnp.int32),
            pltpu.VMEM((CH, D), jnp.float32),
            pltpu.VMEM((CH, D), jnp.float32),
            pltpu.VMEM((ZR, D), jnp.float32),
            pltpu.VMEM_SHARED((NPAD, D), jnp.float32),
            pltpu.SemaphoreType.DMA,
            pltpu.SemaphoreType.DMA,
            pltpu.SemaphoreType.DMA,
            pltpu.SemaphoreType.DMA,
            pltpu.SemaphoreType.DMA,
            pltpu.SemaphoreType.DMA,
        ],
    )
    return deg, msg


def _tc0_body(deg_ref, dinv_ref):
    deg = jnp.sum(deg_ref[...], axis=0) + 1.0
    dinv_ref[...] = lax.rsqrt(deg)[:, None]


def _tc0(deg_parts):
    return pl.pallas_call(
        _tc0_body,
        in_specs=[pl.BlockSpec((NW, N), lambda: (0, 0))],
        out_specs=pl.BlockSpec((N, 1), lambda: (0, 0)),
        out_shape=jax.ShapeDtypeStruct((N, 1), jnp.float32),
    )(deg_parts)


def _tc1_body(dinv_ref, x_ref, w_ref, h_ref):
    h = jnp.dot(x_ref[...], w_ref[...], preferred_element_type=jnp.float32)
    h_ref[...] = h * dinv_ref[...]


def _tc1(dinv, x, W1):
    return pl.pallas_call(
        _tc1_body,
        grid=(NB,),
        in_specs=[
            pl.BlockSpec((BN, 1), lambda i: (i, 0)),
            pl.BlockSpec((BN, D), lambda i: (i, 0)),
            pl.BlockSpec((D, D), lambda i: (0, 0)),
        ],
        out_specs=pl.BlockSpec((BN, D), lambda i: (i, 0)),
        out_shape=jax.ShapeDtypeStruct((N, D), jnp.float32),
    )(dinv, x, W1)


def _onehot(batch_col):
    return (batch_col == lax.broadcasted_iota(jnp.int32, (BN, G), 1)
            ).astype(jnp.float32)


def _stats_body(s_ref, hp_ref, dinv_ref, b_ref, batch_ref,
                z_ref, sums_ref, sq_ref, cnt_ref):
    i = pl.program_id(0)
    z = dinv_ref[...] * (s_ref[0] + s_ref[1] + hp_ref[...]) + b_ref[...]
    z_ref[...] = z
    oh = _onehot(batch_ref[...])
    dn = (((0,), (0,)), ((), ()))
    hi = lax.Precision.HIGHEST
    ssum = lax.dot_general(oh, z, dn, precision=hi,
                           preferred_element_type=jnp.float32)
    ssq = lax.dot_general(oh, z * z, dn, precision=hi,
                          preferred_element_type=jnp.float32)
    scnt = jnp.sum(oh, axis=0)[:, None]

    @pl.when(i == 0)
    def _():
        sums_ref[...] = jnp.zeros_like(sums_ref)
        sq_ref[...] = jnp.zeros_like(sq_ref)
        cnt_ref[...] = jnp.zeros_like(cnt_ref)

    sums_ref[...] += ssum
    sq_ref[...] += ssq
    cnt_ref[...] += scnt


def _tc_stats(s_parts, hp, dinv, b, batch_col):
    return pl.pallas_call(
        _stats_body,
        grid=(NB,),
        in_specs=[
            pl.BlockSpec((NC, BN, D), lambda i: (0, i, 0)),
            pl.BlockSpec((BN, D), lambda i: (i, 0)),
            pl.BlockSpec((BN, 1), lambda i: (i, 0)),
            pl.BlockSpec((1, D), lambda i: (0, 0)),
            pl.BlockSpec((BN, 1), lambda i: (i, 0)),
        ],
        out_specs=[
            pl.BlockSpec((BN, D), lambda i: (i, 0)),
            pl.BlockSpec((G, D), lambda i: (0, 0)),
            pl.BlockSpec((G, D), lambda i: (0, 0)),
            pl.BlockSpec((G, 1), lambda i: (0, 0)),
        ],
        out_shape=[
            jax.ShapeDtypeStruct((N, D), jnp.float32),
            jax.ShapeDtypeStruct((G, D), jnp.float32),
            jax.ShapeDtypeStruct((G, D), jnp.float32),
            jax.ShapeDtypeStruct((G, 1), jnp.float32),
        ],
    )(s_parts, hp, dinv, b.reshape(1, D), batch_col)


def _norm_stats(sums, sq, cnt):
    cntc = jnp.maximum(cnt, 1.0)
    mean = sums / cntc
    var = sq / cntc - mean * mean
    scale = lax.rsqrt(var + EPS)
    return mean, scale


def _tc3_body(z_ref, batch_ref, sums_ref, sq_ref, cnt_ref, dinv_ref, w_ref,
              h2_ref):
    mean, scale = _norm_stats(sums_ref[...], sq_ref[...], cnt_ref[...])
    oh = _onehot(batch_ref[...])
    hi = lax.Precision.HIGHEST
    rmean = jnp.dot(oh, mean, precision=hi,
                    preferred_element_type=jnp.float32)
    rscale = jnp.dot(oh, scale, precision=hi,
                     preferred_element_type=jnp.float32)
    t = jnp.maximum((z_ref[...] - rmean) * rscale, 0.0)
    h2 = jnp.dot(t, w_ref[...], preferred_element_type=jnp.float32)
    h2_ref[...] = h2 * dinv_ref[...]


def _tc3(z, batch_col, sums, sq, cnt, dinv, W2):
    return pl.pallas_call(
        _tc3_body,
        grid=(NB,),
        in_specs=[
            pl.BlockSpec((BN, D), lambda i: (i, 0)),
            pl.BlockSpec((BN, 1), lambda i: (i, 0)),
            pl.BlockSpec((G, D), lambda i: (0, 0)),
            pl.BlockSpec((G, D), lambda i: (0, 0)),
            pl.BlockSpec((G, 1), lambda i: (0, 0)),
            pl.BlockSpec((BN, 1), lambda i: (i, 0)),
            pl.BlockSpec((D, D), lambda i: (0, 0)),
        ],
        out_specs=pl.BlockSpec((BN, D), lambda i: (i, 0)),
        out_shape=jax.ShapeDtypeStruct((N, D), jnp.float32),
    )(z, batch_col, sums, sq, cnt, dinv, W2)


def _tc5_body(z_ref, batch_ref, sums_ref, sq_ref, cnt_ref, fcw_ref, fcb_ref,
              out_ref, pooled):
    i = pl.program_id(0)
    mean, scale = _norm_stats(sums_ref[...], sq_ref[...], cnt_ref[...])
    oh = _onehot(batch_ref[...])
    hi = lax.Precision.HIGHEST
    rmean = jnp.dot(oh, mean, precision=hi,
                    preferred_element_type=jnp.float32)
    rscale = jnp.dot(oh, scale, precision=hi,
                     preferred_element_type=jnp.float32)
    t = jnp.maximum((z_ref[...] - rmean) * rscale, 0.0)
    dn = (((0,), (0,)), ((), ()))
    psum = lax.dot_general(oh, t, dn, precision=hi,
                           preferred_element_type=jnp.float32)

    @pl.when(i == 0)
    def _():
        pooled[...] = jnp.zeros_like(pooled)

    pooled[...] += psum

    @pl.when(i == NB - 1)
    def _():
        cntc = jnp.maximum(cnt_ref[...], 1.0)
        g = pooled[...] / cntc * np.float32(1.0 / np.sqrt(1.0 + EPS))
        out_ref[...] = (jnp.dot(g, fcw_ref[...],
                                preferred_element_type=jnp.float32)
                        + fcb_ref[...])


def _tc5(z2, batch_col, sums2, sq2, cnt, fc_W, fc_b):
    return pl.pallas_call(
        _tc5_body,
        grid=(NB,),
        in_specs=[
            pl.BlockSpec((BN, D), lambda i: (i, 0)),
            pl.BlockSpec((BN, 1), lambda i: (i, 0)),
            pl.BlockSpec((G, D), lambda i: (0, 0)),
            pl.BlockSpec((G, D), lambda i: (0, 0)),
            pl.BlockSpec((G, 1), lambda i: (0, 0)),
            pl.BlockSpec((D, C), lambda i: (0, 0)),
            pl.BlockSpec((1, C), lambda i: (0, 0)),
        ],
        out_specs=pl.BlockSpec((G, C), lambda i: (0, 0)),
        out_shape=jax.ShapeDtypeStruct((G, C), jnp.float32),
        scratch_shapes=[pltpu.VMEM((G, D), jnp.float32)],
    )(z2, batch_col, sums2, sq2, cnt, fc_W, fc_b.reshape(1, C))


def kernel(x, edge_index, batch, W1, b1, W2, b2, fc_W, fc_b):
    ei = edge_index.astype(jnp.int32)
    pad = EPT_P - EPT
    src_p = jnp.pad(ei[0].reshape(NW, EPT), ((0, 0), (0, pad))
                    ).reshape(NW, NP, 2, CH)
    dst_p = jnp.pad(ei[1].reshape(NW, EPT), ((0, 0), (0, pad)),
                    constant_values=NPAD - 1).reshape(NW, NP, 2, CH)
    comb = jnp.concatenate([src_p, dst_p], axis=2)
    dst_r16 = ei[1].reshape(NW, EPT // 16, 16)
    batch_col = batch.astype(jnp.int32).reshape(N, 1)

    deg_kernel, msg_kernel = _sc_kernels()
    deg_parts = deg_kernel(dst_r16).reshape(NW, N)
    dinv = _tc0(deg_parts)
    h1p = _tc1(dinv, x, W1)
    s1 = msg_kernel(h1p, comb)
    z1, sums1, sq1, cnt = _tc_stats(s1, h1p, dinv, b1, batch_col)
    h2p = _tc3(z1, batch_col, sums1, sq1, cnt, dinv, W2)
    s2 = msg_kernel(h2p, comb)
    z2, sums2, sq2, _ = _tc_stats(s2, h2p, dinv, b2, batch_col)
    return _tc5(z2, batch_col, sums2, sq2, cnt, fc_W, fc_b)


# speedup vs baseline: 2.1327x; 2.1327x over previous
import functools

import jax
import jax.numpy as jnp
import numpy as np
from jax import lax
from jax.experimental import pallas as pl
from jax.experimental.pallas import tpu as pltpu
from jax.experimental.pallas import tpu_sc as plsc

N = 10000
E = 320000
D = 128
G = 64
C = 2
EPS = 1e-5

NC = 2
NS = 16
NW = NC * NS
EPT = E // NW
CH = 80
NCH = EPT // CH
NPAD = 10240
RPT = NPAD // NS
ZR = 8

BN = 1000
NB = N // BN


def _deg_body(dst_hbm, out_hbm, dst_v, hist_v):
    c = lax.axis_index("c")
    s = lax.axis_index("s")
    wid = c * NS + s
    pltpu.sync_copy(dst_hbm.at[wid], dst_v)
    z16 = jnp.zeros((16,), jnp.float32)

    def zb(i, carry):
        hist_v[0, pl.ds(i * 16, 16)] = z16
        return carry

    lax.fori_loop(0, N // 16, zb, 0)
    ones16 = jnp.ones((16,), jnp.float32)
    zeros_i = jnp.zeros((16,), jnp.int32)

    def body(i, carry):
        idx = dst_v[i]
        plsc.addupdate_scatter(hist_v, [zeros_i, idx], ones16)
        return carry

    lax.fori_loop(0, EPT // 16, body, 0)
    pltpu.sync_copy(hist_v, out_hbm.at[wid])


def _msg_body(h_hbm, src_hbm, dst_hbm, out_hbm, src_v, dst_v, rows_a,
              rows_b, zbuf, acc_sh, sem_a, sem_b):
    c = lax.axis_index("c")
    s = lax.axis_index("s")
    wid = c * NS + s
    z16 = jnp.zeros((16,), jnp.float32)

    def zb(i, carry):
        zbuf[i // 8, pl.ds((i % 8) * 16, 16)] = z16
        return carry

    lax.fori_loop(0, ZR * (D // 16), zb, 0)

    def za(k, carry):
        pltpu.sync_copy(zbuf, acc_sh.at[pl.ds(s * RPT + k * ZR, ZR)])
        return carry

    lax.fori_loop(0, RPT // ZR, za, 0)
    pltpu.sync_copy(src_hbm.at[wid, 0], src_v)
    pltpu.sync_copy(dst_hbm.at[wid], dst_v)
    plsc.subcore_barrier()

    def body(i, carry):
        ca = 2 * i
        da = pltpu.async_copy(h_hbm.at[src_v.at[pl.ds(ca * CH, CH)]],
                              rows_a, sem_a)
        db = pltpu.async_copy(h_hbm.at[src_v.at[pl.ds(ca * CH + CH, CH)]],
                              rows_b, sem_b)
        da.wait()
        pltpu.sync_copy(rows_a, acc_sh.at[dst_v.at[ca]], add=True)
        db.wait()
        pltpu.sync_copy(rows_b, acc_sh.at[dst_v.at[ca + 1]], add=True)
        return carry

    lax.fori_loop(0, (NCH - 1) // 2, body, 0)
    pltpu.async_copy(h_hbm.at[src_v.at[pl.ds((NCH - 1) * CH, CH)]],
                     rows_a, sem_a).wait()
    pltpu.sync_copy(rows_a, acc_sh.at[dst_v.at[NCH - 1]], add=True)
    plsc.subcore_barrier()
    pltpu.sync_copy(acc_sh.at[pl.ds(s * RPT, RPT)],
                    out_hbm.at[c, pl.ds(s * RPT, RPT)])


@functools.lru_cache(maxsize=None)
def _sc_kernels():
    mesh = plsc.VectorSubcoreMesh(core_axis_name="c", subcore_axis_name="s",
                                  num_cores=NC, num_subcores=NS)
    deg = pl.kernel(
        _deg_body,
        out_type=jax.ShapeDtypeStruct((NW, 1, N), jnp.float32),
        mesh=mesh,
        scratch_types=[
            pltpu.VMEM((EPT // 16, 16), jnp.int32),
            pltpu.VMEM((1, N), jnp.float32),
        ],
        compiler_params=pltpu.CompilerParams(needs_layout_passes=False),
    )
    msg = pl.kernel(
        _msg_body,
        out_type=jax.ShapeDtypeStruct((NC, NPAD, D), jnp.float32),
        mesh=mesh,
        scratch_types=[
            pltpu.VMEM((EPT,), jnp.int32),
            pltpu.VMEM((NCH, CH), jnp.int32),
            pltpu.VMEM((CH, D), jnp.float32),
            pltpu.VMEM((CH, D), jnp.float32),
            pltpu.VMEM((ZR, D), jnp.float32),
            pltpu.VMEM_SHARED((NPAD, D), jnp.float32),
            pltpu.SemaphoreType.DMA,
            pltpu.SemaphoreType.DMA,
        ],
    )
    return deg, msg


def _tc0_body(deg_ref, dinv_ref):
    deg = jnp.sum(deg_ref[...], axis=0) + 1.0
    dinv_ref[...] = lax.rsqrt(deg)[:, None]


def _tc0(deg_parts):
    return pl.pallas_call(
        _tc0_body,
        in_specs=[pl.BlockSpec((NW, N), lambda: (0, 0))],
        out_specs=pl.BlockSpec((N, 1), lambda: (0, 0)),
        out_shape=jax.ShapeDtypeStruct((N, 1), jnp.float32),
    )(deg_parts)


def _tc1_body(dinv_ref, x_ref, w_ref, h_ref):
    h = jnp.dot(x_ref[...], w_ref[...], preferred_element_type=jnp.float32)
    h_ref[...] = h * dinv_ref[...]


def _tc1(dinv, x, W1):
    return pl.pallas_call(
        _tc1_body,
        grid=(NB,),
        in_specs=[
            pl.BlockSpec((BN, 1), lambda i: (i, 0)),
            pl.BlockSpec((BN, D), lambda i: (i, 0)),
            pl.BlockSpec((D, D), lambda i: (0, 0)),
        ],
        out_specs=pl.BlockSpec((BN, D), lambda i: (i, 0)),
        out_shape=jax.ShapeDtypeStruct((N, D), jnp.float32),
    )(dinv, x, W1)


def _onehot(batch_col):
    return (batch_col == lax.broadcasted_iota(jnp.int32, (BN, G), 1)
            ).astype(jnp.float32)


def _stats_body(s_ref, hp_ref, dinv_ref, b_ref, batch_ref,
                z_ref, sums_ref, sq_ref, cnt_ref):
    i = pl.program_id(0)
    z = dinv_ref[...] * (s_ref[0] + s_ref[1] + hp_ref[...]) + b_ref[...]
    z_ref[...] = z
    oh = _onehot(batch_ref[...])
    dn = (((0,), (0,)), ((), ()))
    hi = lax.Precision.HIGHEST
    ssum = lax.dot_general(oh, z, dn, precision=hi,
                           preferred_element_type=jnp.float32)
    ssq = lax.dot_general(oh, z * z, dn, precision=hi,
                          preferred_element_type=jnp.float32)
    scnt = jnp.sum(oh, axis=0)[:, None]

    @pl.when(i == 0)
    def _():
        sums_ref[...] = jnp.zeros_like(sums_ref)
        sq_ref[...] = jnp.zeros_like(sq_ref)
        cnt_ref[...] = jnp.zeros_like(cnt_ref)

    sums_ref[...] += ssum
    sq_ref[...] += ssq
    cnt_ref[...] += scnt


def _tc_stats(s_parts, hp, dinv, b, batch_col):
    return pl.pallas_call(
        _stats_body,
        grid=(NB,),
        in_specs=[
            pl.BlockSpec((NC, BN, D), lambda i: (0, i, 0)),
            pl.BlockSpec((BN, D), lambda i: (i, 0)),
            pl.BlockSpec((BN, 1), lambda i: (i, 0)),
            pl.BlockSpec((1, D), lambda i: (0, 0)),
            pl.BlockSpec((BN, 1), lambda i: (i, 0)),
        ],
        out_specs=[
            pl.BlockSpec((BN, D), lambda i: (i, 0)),
            pl.BlockSpec((G, D), lambda i: (0, 0)),
            pl.BlockSpec((G, D), lambda i: (0, 0)),
            pl.BlockSpec((G, 1), lambda i: (0, 0)),
        ],
        out_shape=[
            jax.ShapeDtypeStruct((N, D), jnp.float32),
            jax.ShapeDtypeStruct((G, D), jnp.float32),
            jax.ShapeDtypeStruct((G, D), jnp.float32),
            jax.ShapeDtypeStruct((G, 1), jnp.float32),
        ],
    )(s_parts, hp, dinv, b.reshape(1, D), batch_col)


def _norm_stats(sums, sq, cnt):
    cntc = jnp.maximum(cnt, 1.0)
    mean = sums / cntc
    var = sq / cntc - mean * mean
    scale = lax.rsqrt(var + EPS)
    return mean, scale


def _tc3_body(z_ref, batch_ref, sums_ref, sq_ref, cnt_ref, dinv_ref, w_ref,
              h2_ref):
    mean, scale = _norm_stats(sums_ref[...], sq_ref[...], cnt_ref[...])
    oh = _onehot(batch_ref[...])
    hi = lax.Precision.HIGHEST
    rmean = jnp.dot(oh, mean, precision=hi,
                    preferred_element_type=jnp.float32)
    rscale = jnp.dot(oh, scale, precision=hi,
                     preferred_element_type=jnp.float32)
    t = jnp.maximum((z_ref[...] - rmean) * rscale, 0.0)
    h2 = jnp.dot(t, w_ref[...], preferred_element_type=jnp.float32)
    h2_ref[...] = h2 * dinv_ref[...]


def _tc3(z, batch_col, sums, sq, cnt, dinv, W2):
    return pl.pallas_call(
        _tc3_body,
        grid=(NB,),
        in_specs=[
            pl.BlockSpec((BN, D), lambda i: (i, 0)),
            pl.BlockSpec((BN, 1), lambda i: (i, 0)),
            pl.BlockSpec((G, D), lambda i: (0, 0)),
            pl.BlockSpec((G, D), lambda i: (0, 0)),
            pl.BlockSpec((G, 1), lambda i: (0, 0)),
            pl.BlockSpec((BN, 1), lambda i: (i, 0)),
            pl.BlockSpec((D, D), lambda i: (0, 0)),
        ],
        out_specs=pl.BlockSpec((BN, D), lambda i: (i, 0)),
        out_shape=jax.ShapeDtypeStruct((N, D), jnp.float32),
    )(z, batch_col, sums, sq, cnt, dinv, W2)


def _tc5_body(z_ref, batch_ref, sums_ref, sq_ref, cnt_ref, fcw_ref, fcb_ref,
              out_ref, pooled):
    i = pl.program_id(0)
    mean, scale = _norm_stats(sums_ref[...], sq_ref[...], cnt_ref[...])
    oh = _onehot(batch_ref[...])
    hi = lax.Precision.HIGHEST
    rmean = jnp.dot(oh, mean, precision=hi,
                    preferred_element_type=jnp.float32)
    rscale = jnp.dot(oh, scale, precision=hi,
                     preferred_element_type=jnp.float32)
    t = jnp.maximum((z_ref[...] - rmean) * rscale, 0.0)
    dn = (((0,), (0,)), ((), ()))
    psum = lax.dot_general(oh, t, dn, precision=hi,
                           preferred_element_type=jnp.float32)

    @pl.when(i == 0)
    def _():
        pooled[...] = jnp.zeros_like(pooled)

    pooled[...] += psum

    @pl.when(i == NB - 1)
    def _():
        cntc = jnp.maximum(cnt_ref[...], 1.0)
        g = pooled[...] / cntc * np.float32(1.0 / np.sqrt(1.0 + EPS))
        out_ref[...] = (jnp.dot(g, fcw_ref[...],
                                preferred_element_type=jnp.float32)
                        + fcb_ref[...])


def _tc5(z2, batch_col, sums2, sq2, cnt, fc_W, fc_b):
    return pl.pallas_call(
        _tc5_body,
        grid=(NB,),
        in_specs=[
            pl.BlockSpec((BN, D), lambda i: (i, 0)),
            pl.BlockSpec((BN, 1), lambda i: (i, 0)),
            pl.BlockSpec((G, D), lambda i: (0, 0)),
            pl.BlockSpec((G, D), lambda i: (0, 0)),
            pl.BlockSpec((G, 1), lambda i: (0, 0)),
            pl.BlockSpec((D, C), lambda i: (0, 0)),
            pl.BlockSpec((1, C), lambda i: (0, 0)),
        ],
        out_specs=pl.BlockSpec((G, C), lambda i: (0, 0)),
        out_shape=jax.ShapeDtypeStruct((G, C), jnp.float32),
        scratch_shapes=[pltpu.VMEM((G, D), jnp.float32)],
    )(z2, batch_col, sums2, sq2, cnt, fc_W, fc_b.reshape(1, C))


def kernel(x, edge_index, batch, W1, b1, W2, b2, fc_W, fc_b):
    ei = edge_index.astype(jnp.int32)
    src_r = ei[0].reshape(NW, 1, EPT)
    dst_r = ei[1].reshape(NW, NCH, CH)
    dst_r16 = ei[1].reshape(NW, EPT // 16, 16)
    batch_col = batch.astype(jnp.int32).reshape(N, 1)

    deg_kernel, msg_kernel = _sc_kernels()
    deg_parts = deg_kernel(dst_r16).reshape(NW, N)
    dinv = _tc0(deg_parts)
    h1p = _tc1(dinv, x, W1)
    s1 = msg_kernel(h1p, src_r, dst_r)
    z1, sums1, sq1, cnt = _tc_stats(s1, h1p, dinv, b1, batch_col)
    h2p = _tc3(z1, batch_col, sums1, sq1, cnt, dinv, W2)
    s2 = msg_kernel(h2p, src_r, dst_r)
    z2, sums2, sq2, _ = _tc_stats(s2, h2p, dinv, b2, batch_col)
    return _tc5(z2, batch_col, sums2, sq2, cnt, fc_W, fc_b)


# speedup vs baseline: 2.1928x; 1.0282x over previous
import functools

import jax
import jax.numpy as jnp
import numpy as np
from jax import lax
from jax.experimental import pallas as pl
from jax.experimental.pallas import tpu as pltpu
from jax.experimental.pallas import tpu_sc as plsc

N = 10000
E = 320000
D = 128
G = 64
C = 2
EPS = 1e-5

NC = 2
NS = 16
NW = NC * NS
EPT = E // NW
CH = 80
NCH = EPT // CH
NPAD = 10240
RPT = NPAD // NS
ZR = 8

BN = 1000
NB = N // BN


def _deg_body(ei_hbm, out_hbm, dst_v, hist_v):
    c = lax.axis_index("c")
    s = lax.axis_index("s")
    wid = c * NS + s
    pltpu.sync_copy(ei_hbm.at[1, wid], dst_v)
    z16 = jnp.zeros((16,), jnp.float32)

    def zb(i, carry):
        hist_v[0, pl.ds(i * 16, 16)] = z16
        return carry

    lax.fori_loop(0, N // 16, zb, 0)
    ones16 = jnp.ones((16,), jnp.float32)
    zeros_i = jnp.zeros((16,), jnp.int32)

    def body(i, carry):
        idx = dst_v[i]
        plsc.addupdate_scatter(hist_v, [zeros_i, idx], ones16)
        return carry

    lax.fori_loop(0, EPT // 16, body, 0)
    pltpu.sync_copy(hist_v, out_hbm.at[wid])


def _msg_body(h_hbm, eif_hbm, eic_hbm, out_hbm, src_v, dst_v, rows_a,
              rows_b, zbuf, acc_sh, sem_a, sem_b, sem_sa, sem_sb):
    c = lax.axis_index("c")
    s = lax.axis_index("s")
    wid = c * NS + s
    z16 = jnp.zeros((16,), jnp.float32)

    def zb(i, carry):
        zbuf[i // 8, pl.ds((i % 8) * 16, 16)] = z16
        return carry

    lax.fori_loop(0, ZR * (D // 16), zb, 0)

    def za(k, carry):
        pltpu.sync_copy(zbuf, acc_sh.at[pl.ds(s * RPT + k * ZR, ZR)])
        return carry

    lax.fori_loop(0, RPT // ZR, za, 0)
    pltpu.sync_copy(eif_hbm.at[0, wid, 0], src_v)
    pltpu.sync_copy(eic_hbm.at[1, wid], dst_v)
    plsc.subcore_barrier()

    def body(i, carry):
        ca = 2 * i
        da = pltpu.async_copy(h_hbm.at[src_v.at[pl.ds(ca * CH, CH)]],
                              rows_a, sem_a)
        db = pltpu.async_copy(h_hbm.at[src_v.at[pl.ds(ca * CH + CH, CH)]],
                              rows_b, sem_b)
        da.wait()
        sa = pltpu.async_copy(rows_a, acc_sh.at[dst_v.at[ca]], sem_sa,
                              add=True)
        db.wait()
        sb = pltpu.async_copy(rows_b, acc_sh.at[dst_v.at[ca + 1]], sem_sb,
                              add=True)
        sa.wait()
        sb.wait()
        return carry

    lax.fori_loop(0, (NCH - 1) // 2, body, 0)
    pltpu.async_copy(h_hbm.at[src_v.at[pl.ds((NCH - 1) * CH, CH)]],
                     rows_a, sem_a).wait()
    pltpu.sync_copy(rows_a, acc_sh.at[dst_v.at[NCH - 1]], add=True)
    plsc.subcore_barrier()
    pltpu.sync_copy(acc_sh.at[pl.ds(s * RPT, RPT)],
                    out_hbm.at[c, pl.ds(s * RPT, RPT)])


@functools.lru_cache(maxsize=None)
def _sc_kernels():
    mesh = plsc.VectorSubcoreMesh(core_axis_name="c", subcore_axis_name="s",
                                  num_cores=NC, num_subcores=NS)
    deg = pl.kernel(
        _deg_body,
        out_type=jax.ShapeDtypeStruct((NW, 1, N), jnp.float32),
        mesh=mesh,
        scratch_types=[
            pltpu.VMEM((EPT // 16, 16), jnp.int32),
            pltpu.VMEM((1, N), jnp.float32),
        ],
        compiler_params=pltpu.CompilerParams(needs_layout_passes=False),
    )
    msg = pl.kernel(
        _msg_body,
        out_type=jax.ShapeDtypeStruct((NC, NPAD, D), jnp.float32),
        mesh=mesh,
        scratch_types=[
            pltpu.VMEM((EPT,), jnp.int32),
            pltpu.VMEM((NCH, CH), jnp.int32),
            pltpu.VMEM((CH, D), jnp.float32),
            pltpu.VMEM((CH, D), jnp.float32),
            pltpu.VMEM((ZR, D), jnp.float32),
            pltpu.VMEM_SHARED((NPAD, D), jnp.float32),
            pltpu.SemaphoreType.DMA,
            pltpu.SemaphoreType.DMA,
            pltpu.SemaphoreType.DMA,
            pltpu.SemaphoreType.DMA,
        ],
    )
    return deg, msg


def _tc0_body(deg_ref, dinv_ref):
    deg = jnp.sum(deg_ref[...][:, 0, :], axis=0) + 1.0
    dinv_ref[...] = lax.rsqrt(deg)[:, None]


def _tc0(deg_parts):
    return pl.pallas_call(
        _tc0_body,
        in_specs=[pl.BlockSpec((NW, 1, N), lambda: (0, 0, 0))],
        out_specs=pl.BlockSpec((N, 1), lambda: (0, 0)),
        out_shape=jax.ShapeDtypeStruct((N, 1), jnp.float32),
    )(deg_parts)


def _tc1_body(dinv_ref, x_ref, w_ref, h_ref):
    h = jnp.dot(x_ref[...], w_ref[...], preferred_element_type=jnp.float32)
    h_ref[...] = h * dinv_ref[...]


def _tc1(dinv, x, W1):
    return pl.pallas_call(
        _tc1_body,
        grid=(NB,),
        in_specs=[
            pl.BlockSpec((BN, 1), lambda i: (i, 0)),
            pl.BlockSpec((BN, D), lambda i: (i, 0)),
            pl.BlockSpec((D, D), lambda i: (0, 0)),
        ],
        out_specs=pl.BlockSpec((BN, D), lambda i: (i, 0)),
        out_shape=jax.ShapeDtypeStruct((N, D), jnp.float32),
    )(dinv, x, W1)


def _onehot(batch_col):
    return (batch_col == lax.broadcasted_iota(jnp.int32, (BN, G), 1)
            ).astype(jnp.float32)


def _stats_body(s_ref, hp_ref, dinv_ref, b_ref, batch_ref,
                z_ref, sums_ref, sq_ref, cnt_ref):
    i = pl.program_id(0)
    z = dinv_ref[...] * (s_ref[0] + s_ref[1] + hp_ref[...]) + b_ref[...]
    z_ref[...] = z
    oh = _onehot(batch_ref[...])
    dn = (((0,), (0,)), ((), ()))
    hi = lax.Precision.HIGHEST
    ssum = lax.dot_general(oh, z, dn, precision=hi,
                           preferred_element_type=jnp.float32)
    ssq = lax.dot_general(oh, z * z, dn, precision=hi,
                          preferred_element_type=jnp.float32)
    scnt = jnp.sum(oh, axis=0)[:, None]

    @pl.when(i == 0)
    def _():
        sums_ref[...] = jnp.zeros_like(sums_ref)
        sq_ref[...] = jnp.zeros_like(sq_ref)
        cnt_ref[...] = jnp.zeros_like(cnt_ref)

    sums_ref[...] += ssum
    sq_ref[...] += ssq
    cnt_ref[...] += scnt


def _tc_stats(s_parts, hp, dinv, b, batch_col):
    return pl.pallas_call(
        _stats_body,
        grid=(NB,),
        in_specs=[
            pl.BlockSpec((NC, BN, D), lambda i: (0, i, 0)),
            pl.BlockSpec((BN, D), lambda i: (i, 0)),
            pl.BlockSpec((BN, 1), lambda i: (i, 0)),
            pl.BlockSpec((1, D), lambda i: (0, 0)),
            pl.BlockSpec((BN, 1), lambda i: (i, 0)),
        ],
        out_specs=[
            pl.BlockSpec((BN, D), lambda i: (i, 0)),
            pl.BlockSpec((G, D), lambda i: (0, 0)),
            pl.BlockSpec((G, D), lambda i: (0, 0)),
            pl.BlockSpec((G, 1), lambda i: (0, 0)),
        ],
        out_shape=[
            jax.ShapeDtypeStruct((N, D), jnp.float32),
            jax.ShapeDtypeStruct((G, D), jnp.float32),
            jax.ShapeDtypeStruct((G, D), jnp.float32),
            jax.ShapeDtypeStruct((G, 1), jnp.float32),
        ],
    )(s_parts, hp, dinv, b.reshape(1, D), batch_col)


def _norm_stats(sums, sq, cnt):
    cntc = jnp.maximum(cnt, 1.0)
    mean = sums / cntc
    var = sq / cntc - mean * mean
    scale = lax.rsqrt(var + EPS)
    return mean, scale


def _tc3_body(z_ref, batch_ref, sums_ref, sq_ref, cnt_ref, dinv_ref, w_ref,
              h2_ref):
    mean, scale = _norm_stats(sums_ref[...], sq_ref[...], cnt_ref[...])
    oh = _onehot(batch_ref[...])
    hi = lax.Precision.HIGHEST
    rmean = jnp.dot(oh, mean, precision=hi,
                    preferred_element_type=jnp.float32)
    rscale = jnp.dot(oh, scale, precision=hi,
                     preferred_element_type=jnp.float32)
    t = jnp.maximum((z_ref[...] - rmean) * rscale, 0.0)
    h2 = jnp.dot(t, w_ref[...], preferred_element_type=jnp.float32)
    h2_ref[...] = h2 * dinv_ref[...]


def _tc3(z, batch_col, sums, sq, cnt, dinv, W2):
    return pl.pallas_call(
        _tc3_body,
        grid=(NB,),
        in_specs=[
            pl.BlockSpec((BN, D), lambda i: (i, 0)),
            pl.BlockSpec((BN, 1), lambda i: (i, 0)),
            pl.BlockSpec((G, D), lambda i: (0, 0)),
            pl.BlockSpec((G, D), lambda i: (0, 0)),
            pl.BlockSpec((G, 1), lambda i: (0, 0)),
            pl.BlockSpec((BN, 1), lambda i: (i, 0)),
            pl.BlockSpec((D, D), lambda i: (0, 0)),
        ],
        out_specs=pl.BlockSpec((BN, D), lambda i: (i, 0)),
        out_shape=jax.ShapeDtypeStruct((N, D), jnp.float32),
    )(z, batch_col, sums, sq, cnt, dinv, W2)


def _tc5_body(z_ref, batch_ref, sums_ref, sq_ref, cnt_ref, fcw_ref, fcb_ref,
              out_ref, pooled):
    i = pl.program_id(0)
    mean, scale = _norm_stats(sums_ref[...], sq_ref[...], cnt_ref[...])
    oh = _onehot(batch_ref[...])
    hi = lax.Precision.HIGHEST
    rmean = jnp.dot(oh, mean, precision=hi,
                    preferred_element_type=jnp.float32)
    rscale = jnp.dot(oh, scale, precision=hi,
                     preferred_element_type=jnp.float32)
    t = jnp.maximum((z_ref[...] - rmean) * rscale, 0.0)
    dn = (((0,), (0,)), ((), ()))
    psum = lax.dot_general(oh, t, dn, precision=hi,
                           preferred_element_type=jnp.float32)

    @pl.when(i == 0)
    def _():
        pooled[...] = jnp.zeros_like(pooled)

    pooled[...] += psum

    @pl.when(i == NB - 1)
    def _():
        cntc = jnp.maximum(cnt_ref[...], 1.0)
        g = pooled[...] / cntc * np.float32(1.0 / np.sqrt(1.0 + EPS))
        out_ref[...] = (jnp.dot(g, fcw_ref[...],
                                preferred_element_type=jnp.float32)
                        + fcb_ref[...])


def _tc5(z2, batch_col, sums2, sq2, cnt, fc_W, fc_b):
    return pl.pallas_call(
        _tc5_body,
        grid=(NB,),
        in_specs=[
            pl.BlockSpec((BN, D), lambda i: (i, 0)),
            pl.BlockSpec((BN, 1), lambda i: (i, 0)),
            pl.BlockSpec((G, D), lambda i: (0, 0)),
            pl.BlockSpec((G, D), lambda i: (0, 0)),
            pl.BlockSpec((G, 1), lambda i: (0, 0)),
            pl.BlockSpec((D, C), lambda i: (0, 0)),
            pl.BlockSpec((1, C), lambda i: (0, 0)),
        ],
        out_specs=pl.BlockSpec((G, C), lambda i: (0, 0)),
        out_shape=jax.ShapeDtypeStruct((G, C), jnp.float32),
        scratch_shapes=[pltpu.VMEM((G, D), jnp.float32)],
    )(z2, batch_col, sums2, sq2, cnt, fc_W, fc_b.reshape(1, C))


def kernel(x, edge_index, batch, W1, b1, W2, b2, fc_W, fc_b):
    ei = edge_index.astype(jnp.int32)
    ei_a = ei.reshape(2, NW, EPT // 16, 16)
    ei_f = ei.reshape(2, NW, 1, EPT)
    ei_c = ei.reshape(2, NW, NCH, CH)
    batch_col = batch.astype(jnp.int32).reshape(N, 1)

    deg_kernel, msg_kernel = _sc_kernels()
    deg_parts = deg_kernel(ei_a)
    dinv = _tc0(deg_parts)
    h1p = _tc1(dinv, x, W1)
    s1 = msg_kernel(h1p, ei_f, ei_c)
    z1, sums1, sq1, cnt = _tc_stats(s1, h1p, dinv, b1, batch_col)
    h2p = _tc3(z1, batch_col, sums1, sq1, cnt, dinv, W2)
    s2 = msg_kernel(h2p, ei_f, ei_c)
    z2, sums2, sq2, _ = _tc_stats(s2, h2p, dinv, b2, batch_col)
    return _tc5(z2, batch_col, sums2, sq2, cnt, fc_W, fc_b)
